# baseline (speedup 1.0000x reference)
import functools

import numpy as np
import jax
import jax.numpy as jnp
from jax import lax
from jax.experimental import pallas as pl
from jax.experimental.pallas import tpu as pltpu
from jax.experimental.pallas import tpu_sc as plsc

_N = 100000
_FEAT = 128
_NH = 4
_O = 16
_HD = _NH * _O
_S = 10
_B = 1024
_NMP = 2
_GW = 128
_IDC = 80

_NC, _NS = 2, 16
_NW = _NC * _NS


def _pretransform(feats, adjs, wext):
    n = feats.shape[0]
    bn = 2000

    def body(x_ref, a_ref, w_ref, o_ref):
        y = jnp.dot(x_ref[...], w_ref[...], preferred_element_type=jnp.float32)
        z = jnp.zeros((bn, _GW - _IDC - _S), jnp.float32)
        for mp in range(_NMP):
            idsf = lax.bitcast_convert_type(
                a_ref[mp, :, :_S] + jnp.int32(mp * _N), jnp.float32)
            o_ref[mp] = jnp.concatenate(
                [y[:, mp * 80:(mp + 1) * 80], idsf, z], axis=1)

    return pl.pallas_call(
        body,
        grid=(n // bn,),
        in_specs=[pl.BlockSpec((bn, _FEAT), lambda i: (i, 0)),
                  pl.BlockSpec((_NMP, bn, 32), lambda i: (0, i, 0)),
                  pl.BlockSpec((_FEAT, 160), lambda i: (0, 0))],
        out_specs=pl.BlockSpec((_NMP, bn, _GW), lambda i: (0, i, 0)),
        out_shape=jax.ShapeDtypeStruct((_NMP, n, _GW), jnp.float32),
    )(feats, adjs, wext)


_NRING = 5


def _sc_sample_gather(HX, idsx):
    n_seed = 64
    iota = lambda: lax.broadcasted_iota(jnp.int32, (16,), 0)

    def extract(src, col, n_rows, dst, dst_off):
        for i in range(n_rows // 16):
            v = plsc.load_gather(src, [i * 16 + iota(),
                                       jnp.full((16,), col, jnp.int32)])
            dst[pl.ds(dst_off + i * 16, 16)] = plsc.bitcast(v, jnp.int32)

    def body(hx_hbm, ids_hbm, g0_hbm, g1_hbm, g2_hbm,
             seedbuf, l1buf, l2buf, ring, asem, *rs):
        wid = lax.axis_index("s") * _NC + lax.axis_index("c")
        pltpu.sync_copy(ids_hbm.at[pl.ds(wid * n_seed, n_seed)], seedbuf)
        pltpu.async_copy(hx_hbm.at[seedbuf], ring.at[0, pl.ds(0, n_seed)],
                         asem).wait()
        pltpu.sync_copy(ring.at[0, pl.ds(0, n_seed)],
                        g0_hbm.at[pl.ds(wid * n_seed, n_seed)])
        for c1 in range(_S):
            extract(ring.at[0, pl.ds(0, n_seed)], _IDC + c1, n_seed,
                    l1buf, c1 * n_seed)

        for ch in range(_S * n_seed // 128):
            pltpu.async_copy(hx_hbm.at[l1buf.at[pl.ds(ch * 128, 128)]],
                             ring.at[0], asem).wait()
            for half in range(2):
                c1 = 2 * ch + half
                pltpu.sync_copy(
                    ring.at[0, pl.ds(half * 64, 64)],
                    g1_hbm.at[pl.ds(c1 * (_NMP * _B) + wid * n_seed, 64)])
                for s2 in range(_S):
                    extract(ring.at[0, pl.ds(half * 64, 64)], _IDC + s2,
                            n_seed, l2buf, s2 * 640 + c1 * n_seed)

        nch2 = _S * _S * n_seed // 128

        def l2desc(ch, b):
            return pltpu.make_async_copy(
                hx_hbm.at[l2buf.at[pl.ds(ch * 128, 128)]], ring.at[b], rs[b])

        for b in range(_NRING):
            l2desc(b, b).start()

        def step(j, carry):
            for b in range(_NRING):
                ch = j * _NRING + b
                l2desc(ch, b).wait()
                p0 = ch * 128
                s2 = p0 // 640
                c1 = (p0 - s2 * 640) // 64
                for half in range(2):
                    dst = ((s2 + (c1 + half) // _S) * (_S * _NMP * _B)
                           + ((c1 + half) % _S) * (_NMP * _B) + wid * n_seed)
                    pltpu.sync_copy(ring.at[b, pl.ds(half * 64, 64)],
                                    g2_hbm.at[pl.ds(dst, 64)])

                @pl.when(ch + _NRING < nch2)
                def _():
                    l2desc(ch + _NRING, b).start()
            return carry

        lax.fori_loop(0, nch2 // _NRING, step, jnp.int32(0))

    mesh = plsc.VectorSubcoreMesh(core_axis_name="c", subcore_axis_name="s")
    f = pl.kernel(
        body,
        out_type=(jax.ShapeDtypeStruct((_NMP * _B, _GW), jnp.float32),
                  jax.ShapeDtypeStruct((_S * _NMP * _B, _GW), jnp.float32),
                  jax.ShapeDtypeStruct((_S * _S * _NMP * _B, _GW),
                                       jnp.float32)),
        mesh=mesh,
        compiler_params=pltpu.CompilerParams(needs_layout_passes=False),
        scratch_types=[pltpu.VMEM((n_seed,), jnp.int32),
                       pltpu.VMEM((_S * n_seed,), jnp.int32),
                       pltpu.VMEM((_S * _S * n_seed,), jnp.int32),
                       pltpu.VMEM((_NRING, 128, _GW), jnp.float32),
                       pltpu.SemaphoreType.DMA]
                      + [pltpu.SemaphoreType.DMA] * _NRING,
    )
    return f(HX, idsx)


def _head_mats():
    m = np.zeros((_NH, _HD), np.float32)
    for h in range(_NH):
        m[h, h * _O:(h + 1) * _O] = 1.0
    return jnp.asarray(m)


def _agg_math(sg, ng, hmat):
    e_n = ng[:, :, _HD:_HD + _NH]
    e_s = sg[:, _HD + _NH:_HD + 2 * _NH]
    sc = e_n + e_s[None]
    sc = jnp.where(sc >= 0, sc, 0.2 * sc)
    ex = jnp.exp(sc)
    den = jnp.sum(ex, axis=0)
    dot = functools.partial(jnp.dot, preferred_element_type=jnp.float32)
    acc = jnp.zeros((sg.shape[0], _HD), jnp.float32)
    for s in range(_S):
        acc = acc + dot(ex[s], hmat) * ng[s, :, :_HD]
    return jnp.maximum(sg[:, :_HD] + acc / dot(den, hmat), 0.0)


def _agg1(self_g, neigh_g):
    n = self_g.shape[1]
    nb = min(n, 512)

    def body(s_ref, g_ref, h_ref, o_ref):
        o_ref[0] = _agg_math(s_ref[0], g_ref[:, 0], h_ref[...])

    return pl.pallas_call(
        body,
        grid=(_NMP, n // nb),
        in_specs=[pl.BlockSpec((1, nb, _GW), lambda p, i: (p, i, 0)),
                  pl.BlockSpec((_S, 1, nb, _GW), lambda p, i: (0, p, i, 0)),
                  pl.BlockSpec((_NH, _HD), lambda p, i: (0, 0))],
        out_specs=pl.BlockSpec((1, nb, _HD), lambda p, i: (p, i, 0)),
        out_shape=jax.ShapeDtypeStruct((_NMP, n, _HD), jnp.float32),
    )(self_g, neigh_g, _head_mats())


def _agg_mid(self_g, neigh_g):
    nb = 512

    def body(s_ref, g_ref, h_ref, o_ref):
        o_ref[0, 0] = _agg_math(s_ref[0, 0], g_ref[:, 0, 0], h_ref[...])

    return pl.pallas_call(
        body,
        grid=(_S, _NMP, _B // nb),
        in_specs=[pl.BlockSpec((1, 1, nb, _GW), lambda c, p, i: (c, p, i, 0)),
                  pl.BlockSpec((_S, 1, 1, nb, _GW),
                               lambda c, p, i: (0, c, p, i, 0)),
                  pl.BlockSpec((_NH, _HD), lambda c, p, i: (0, 0))],
        out_specs=pl.BlockSpec((1, 1, nb, _HD), lambda c, p, i: (c, p, i, 0)),
        out_shape=jax.ShapeDtypeStruct((_S, _NMP, _B, _HD), jnp.float32),
    )(self_g, neigh_g, _head_mats())


def _agg_fin(self_g, neigh_g):
    nb = 512

    def body(s_ref, g_ref, h_ref, o_ref):
        o_ref[0] = _agg_math(s_ref[0], g_ref[:, 0], h_ref[...])

    return pl.pallas_call(
        body,
        grid=(_NMP, _B // nb),
        in_specs=[pl.BlockSpec((1, nb, 80), lambda p, i: (p, i, 0)),
                  pl.BlockSpec((_S, 1, nb, 80), lambda p, i: (0, p, i, 0)),
                  pl.BlockSpec((_NH, _HD), lambda p, i: (0, 0))],
        out_specs=pl.BlockSpec((1, nb, _HD), lambda p, i: (p, i, 0)),
        out_shape=jax.ShapeDtypeStruct((_NMP, _B, _HD), jnp.float32),
    )(self_g, neigh_g, _head_mats())


def _mm0(x, w):
    def body(x_ref, w_ref, o_ref):
        o_ref[0] = jnp.dot(x_ref[0], w_ref[0],
                           preferred_element_type=jnp.float32)

    return pl.pallas_call(
        body,
        grid=(_NMP,),
        in_specs=[pl.BlockSpec((1, _B, _HD), lambda p: (p, 0, 0)),
                  pl.BlockSpec((1, _HD, 80), lambda p: (p, 0, 0))],
        out_specs=pl.BlockSpec((1, _B, 80), lambda p: (p, 0, 0)),
        out_shape=jax.ShapeDtypeStruct((_NMP, _B, 80), jnp.float32))(x, w)


def _mm1(x, w):
    def body(x_ref, w_ref, o_ref):
        o_ref[0, 0] = jnp.dot(x_ref[0, 0], w_ref[0],
                              preferred_element_type=jnp.float32)

    return pl.pallas_call(
        body,
        grid=(_S, _NMP),
        in_specs=[pl.BlockSpec((1, 1, _B, _HD), lambda c, p: (c, p, 0, 0)),
                  pl.BlockSpec((1, _HD, 80), lambda c, p: (p, 0, 0))],
        out_specs=pl.BlockSpec((1, 1, _B, 80), lambda c, p: (c, p, 0, 0)),
        out_shape=jax.ShapeDtypeStruct((_S, _NMP, _B, 80), jnp.float32))(x, w)


def _fold(W, a_s, a_n):
    d = W.shape[2]
    heads = jnp.transpose(W, (0, 2, 1, 3)).reshape(_NMP, d, _HD)
    en = jnp.einsum("mhdo,mho->mdh", W, a_n)
    es = jnp.einsum("mhdo,mho->mdh", W, a_s)
    pad = jnp.zeros((_NMP, d, 80 - _HD - 2 * _NH), jnp.float32)
    return jnp.concatenate([heads, en, es, pad], axis=2)


def kernel(ids, feats, adjs, W0, a0_self, a0_neigh, W1, a1_self, a1_neigh):
    w0ext = _fold(W0, a0_self, a0_neigh)
    w1ext = _fold(W1, a1_self, a1_neigh)
    w0cat = jnp.transpose(w0ext, (1, 0, 2)).reshape(_FEAT, _NMP * 80)

    HX = _pretransform(feats, adjs, w0cat)
    HXf = HX.reshape(_NMP * _N, _GW)

    mp_off = (jnp.arange(_NMP, dtype=jnp.int32) * _N)[:, None]
    idsx = (jnp.broadcast_to(ids[None], (_NMP, _B)) + mp_off).reshape(-1)

    g0, g1, g2 = _sc_sample_gather(HXf, idsx)
    g0 = g0.reshape(_NMP, _B, _GW)
    g1 = g1.reshape(_S, _NMP, _B, _GW)
    g2 = g2.reshape(_S, _S, _NMP, _B, _GW)

    out1 = _agg_mid(g1, g2)
    out0 = _agg1(g0, g1)

    gt0 = _mm0(out0, w1ext)
    gt1 = _mm1(out1, w1ext)
    return _agg_fin(gt0, gt1)


# speedup vs baseline: 1.0257x; 1.0257x over previous
import functools

import numpy as np
import jax
import jax.numpy as jnp
from jax import lax
from jax.experimental import pallas as pl
from jax.experimental.pallas import tpu as pltpu
from jax.experimental.pallas import tpu_sc as plsc

_N = 100000
_FEAT = 128
_NH = 4
_O = 16
_HD = _NH * _O
_S = 10
_B = 1024
_NMP = 2
_GW = 128
_IDC = 80

_NC, _NS = 2, 16
_NW = _NC * _NS


def _pretransform(feats, adjs, wext):
    n = feats.shape[0]
    bn = 5000

    def body(x_ref, a_ref, w_ref, o_ref):
        y = jnp.dot(x_ref[...], w_ref[...], preferred_element_type=jnp.float32)
        z = jnp.zeros((bn, _GW - _IDC - _S), jnp.float32)
        for mp in range(_NMP):
            idsf = lax.bitcast_convert_type(
                a_ref[mp, :, :_S] + jnp.int32(mp * _N), jnp.float32)
            o_ref[mp] = jnp.concatenate(
                [y[:, mp * 80:(mp + 1) * 80], idsf, z], axis=1)

    return pl.pallas_call(
        body,
        grid=(n // bn,),
        in_specs=[pl.BlockSpec((bn, _FEAT), lambda i: (i, 0)),
                  pl.BlockSpec((_NMP, bn, 32), lambda i: (0, i, 0)),
                  pl.BlockSpec((_FEAT, 160), lambda i: (0, 0))],
        out_specs=pl.BlockSpec((_NMP, bn, _GW), lambda i: (0, i, 0)),
        out_shape=jax.ShapeDtypeStruct((_NMP, n, _GW), jnp.float32),
    )(feats, adjs, wext)


_NRING = 5


def _sc_sample_gather(HX, idsx):
    n_seed = 64
    iota = lambda: lax.broadcasted_iota(jnp.int32, (16,), 0)

    def extract(src, col, n_rows, dst, dst_off):
        for i in range(n_rows // 16):
            v = plsc.load_gather(src, [i * 16 + iota(),
                                       jnp.full((16,), col, jnp.int32)])
            dst[pl.ds(dst_off + i * 16, 16)] = plsc.bitcast(v, jnp.int32)

    def body(hx_hbm, ids_hbm, g0_hbm, g1_hbm, g2_hbm,
             seedbuf, l1buf, l2buf, ring, asem, *rs):
        wid = lax.axis_index("s") * _NC + lax.axis_index("c")
        pltpu.sync_copy(ids_hbm.at[pl.ds(wid * n_seed, n_seed)], seedbuf)
        pltpu.async_copy(hx_hbm.at[seedbuf], ring.at[0, pl.ds(0, n_seed)],
                         asem).wait()
        pltpu.sync_copy(ring.at[0, pl.ds(0, n_seed)],
                        g0_hbm.at[pl.ds(wid * n_seed, n_seed)])
        for c1 in range(_S):
            extract(ring.at[0, pl.ds(0, n_seed)], _IDC + c1, n_seed,
                    l1buf, c1 * n_seed)

        for ch in range(_S * n_seed // 128):
            pltpu.async_copy(hx_hbm.at[l1buf.at[pl.ds(ch * 128, 128)]],
                             ring.at[0], asem).wait()
            for half in range(2):
                c1 = 2 * ch + half
                pltpu.sync_copy(
                    ring.at[0, pl.ds(half * 64, 64)],
                    g1_hbm.at[pl.ds(c1 * (_NMP * _B) + wid * n_seed, 64)])
                for s2 in range(_S):
                    extract(ring.at[0, pl.ds(half * 64, 64)], _IDC + s2,
                            n_seed, l2buf, s2 * 640 + c1 * n_seed)

        nch2 = _S * _S * n_seed // 128

        def l2desc(ch, b):
            return pltpu.make_async_copy(
                hx_hbm.at[l2buf.at[pl.ds(ch * 128, 128)]], ring.at[b], rs[b])

        for b in range(_NRING):
            l2desc(b, b).start()

        def step(j, carry):
            for b in range(_NRING):
                ch = j * _NRING + b
                l2desc(ch, b).wait()
                p0 = ch * 128
                s2 = p0 // 640
                c1 = (p0 - s2 * 640) // 64
                for half in range(2):
                    dst = ((s2 + (c1 + half) // _S) * (_S * _NMP * _B)
                           + ((c1 + half) % _S) * (_NMP * _B) + wid * n_seed)
                    pltpu.sync_copy(ring.at[b, pl.ds(half * 64, 64)],
                                    g2_hbm.at[pl.ds(dst, 64)])

                @pl.when(ch + _NRING < nch2)
                def _():
                    l2desc(ch + _NRING, b).start()
            return carry

        lax.fori_loop(0, nch2 // _NRING, step, jnp.int32(0))

    mesh = plsc.VectorSubcoreMesh(core_axis_name="c", subcore_axis_name="s")
    f = pl.kernel(
        body,
        out_type=(jax.ShapeDtypeStruct((_NMP * _B, _GW), jnp.float32),
                  jax.ShapeDtypeStruct((_S * _NMP * _B, _GW), jnp.float32),
                  jax.ShapeDtypeStruct((_S * _S * _NMP * _B, _GW),
                                       jnp.float32)),
        mesh=mesh,
        compiler_params=pltpu.CompilerParams(needs_layout_passes=False),
        scratch_types=[pltpu.VMEM((n_seed,), jnp.int32),
                       pltpu.VMEM((_S * n_seed,), jnp.int32),
                       pltpu.VMEM((_S * _S * n_seed,), jnp.int32),
                       pltpu.VMEM((_NRING, 128, _GW), jnp.float32),
                       pltpu.SemaphoreType.DMA]
                      + [pltpu.SemaphoreType.DMA] * _NRING,
    )
    return f(HX, idsx)


def _head_mats():
    m = np.zeros((_NH, _HD), np.float32)
    for h in range(_NH):
        m[h, h * _O:(h + 1) * _O] = 1.0
    return jnp.asarray(m)


def _agg_math(sg, ng, hmat):
    e_n = ng[:, :, _HD:_HD + _NH]
    e_s = sg[:, _HD + _NH:_HD + 2 * _NH]
    sc = e_n + e_s[None]
    sc = jnp.where(sc >= 0, sc, 0.2 * sc)
    ex = jnp.exp(sc)
    den = jnp.sum(ex, axis=0)
    dot = functools.partial(jnp.dot, preferred_element_type=jnp.float32)
    acc = jnp.zeros((sg.shape[0], _HD), jnp.float32)
    for s in range(_S):
        acc = acc + dot(ex[s], hmat) * ng[s, :, :_HD]
    return jnp.maximum(sg[:, :_HD] + acc / dot(den, hmat), 0.0)


def _agg1(self_g, neigh_g):
    n = self_g.shape[1]
    nb = min(n, 512)

    def body(s_ref, g_ref, h_ref, o_ref):
        o_ref[0] = _agg_math(s_ref[0], g_ref[:, 0], h_ref[...])

    return pl.pallas_call(
        body,
        grid=(_NMP, n // nb),
        in_specs=[pl.BlockSpec((1, nb, _GW), lambda p, i: (p, i, 0)),
                  pl.BlockSpec((_S, 1, nb, _GW), lambda p, i: (0, p, i, 0)),
                  pl.BlockSpec((_NH, _HD), lambda p, i: (0, 0))],
        out_specs=pl.BlockSpec((1, nb, _HD), lambda p, i: (p, i, 0)),
        out_shape=jax.ShapeDtypeStruct((_NMP, n, _HD), jnp.float32),
    )(self_g, neigh_g, _head_mats())


def _agg_mid(self_g, neigh_g):
    nb = 512

    def body(s_ref, g_ref, h_ref, o_ref):
        o_ref[0, 0] = _agg_math(s_ref[0, 0], g_ref[:, 0, 0], h_ref[...])

    return pl.pallas_call(
        body,
        grid=(_S, _NMP, _B // nb),
        in_specs=[pl.BlockSpec((1, 1, nb, _GW), lambda c, p, i: (c, p, i, 0)),
                  pl.BlockSpec((_S, 1, 1, nb, _GW),
                               lambda c, p, i: (0, c, p, i, 0)),
                  pl.BlockSpec((_NH, _HD), lambda c, p, i: (0, 0))],
        out_specs=pl.BlockSpec((1, 1, nb, _HD), lambda c, p, i: (c, p, i, 0)),
        out_shape=jax.ShapeDtypeStruct((_S, _NMP, _B, _HD), jnp.float32),
    )(self_g, neigh_g, _head_mats())


def _agg_fin(self_g, neigh_g):
    nb = 512

    def body(s_ref, g_ref, h_ref, o_ref):
        o_ref[0] = _agg_math(s_ref[0], g_ref[:, 0], h_ref[...])

    return pl.pallas_call(
        body,
        grid=(_NMP, _B // nb),
        in_specs=[pl.BlockSpec((1, nb, 80), lambda p, i: (p, i, 0)),
                  pl.BlockSpec((_S, 1, nb, 80), lambda p, i: (0, p, i, 0)),
                  pl.BlockSpec((_NH, _HD), lambda p, i: (0, 0))],
        out_specs=pl.BlockSpec((1, nb, _HD), lambda p, i: (p, i, 0)),
        out_shape=jax.ShapeDtypeStruct((_NMP, _B, _HD), jnp.float32),
    )(self_g, neigh_g, _head_mats())


def _mm0(x, w):
    def body(x_ref, w_ref, o_ref):
        o_ref[0] = jnp.dot(x_ref[0], w_ref[0],
                           preferred_element_type=jnp.float32)

    return pl.pallas_call(
        body,
        grid=(_NMP,),
        in_specs=[pl.BlockSpec((1, _B, _HD), lambda p: (p, 0, 0)),
                  pl.BlockSpec((1, _HD, 80), lambda p: (p, 0, 0))],
        out_specs=pl.BlockSpec((1, _B, 80), lambda p: (p, 0, 0)),
        out_shape=jax.ShapeDtypeStruct((_NMP, _B, 80), jnp.float32))(x, w)


def _mm1(x, w):
    def body(x_ref, w_ref, o_ref):
        o_ref[0, 0] = jnp.dot(x_ref[0, 0], w_ref[0],
                              preferred_element_type=jnp.float32)

    return pl.pallas_call(
        body,
        grid=(_S, _NMP),
        in_specs=[pl.BlockSpec((1, 1, _B, _HD), lambda c, p: (c, p, 0, 0)),
                  pl.BlockSpec((1, _HD, 80), lambda c, p: (p, 0, 0))],
        out_specs=pl.BlockSpec((1, 1, _B, 80), lambda c, p: (c, p, 0, 0)),
        out_shape=jax.ShapeDtypeStruct((_S, _NMP, _B, 80), jnp.float32))(x, w)


def _fold(W, a_s, a_n):
    d = W.shape[2]
    heads = jnp.transpose(W, (0, 2, 1, 3)).reshape(_NMP, d, _HD)
    en = jnp.einsum("mhdo,mho->mdh", W, a_n)
    es = jnp.einsum("mhdo,mho->mdh", W, a_s)
    pad = jnp.zeros((_NMP, d, 80 - _HD - 2 * _NH), jnp.float32)
    return jnp.concatenate([heads, en, es, pad], axis=2)


def kernel(ids, feats, adjs, W0, a0_self, a0_neigh, W1, a1_self, a1_neigh):
    w0ext = _fold(W0, a0_self, a0_neigh)
    w1ext = _fold(W1, a1_self, a1_neigh)
    w0cat = jnp.transpose(w0ext, (1, 0, 2)).reshape(_FEAT, _NMP * 80)

    HX = _pretransform(feats, adjs, w0cat)
    HXf = HX.reshape(_NMP * _N, _GW)

    mp_off = (jnp.arange(_NMP, dtype=jnp.int32) * _N)[:, None]
    idsx = (jnp.broadcast_to(ids[None], (_NMP, _B)) + mp_off).reshape(-1)

    g0, g1, g2 = _sc_sample_gather(HXf, idsx)
    g0 = g0.reshape(_NMP, _B, _GW)
    g1 = g1.reshape(_S, _NMP, _B, _GW)
    g2 = g2.reshape(_S, _S, _NMP, _B, _GW)

    out1 = _agg_mid(g1, g2)
    out0 = _agg1(g0, g1)

    gt0 = _mm0(out0, w1ext)
    gt1 = _mm1(out1, w1ext)
    return _agg_fin(gt0, gt1)
